# baseline (speedup 1.0000x reference)
import functools

import jax
import jax.numpy as jnp
from jax import lax
from jax.experimental import pallas as pl
from jax.experimental.pallas import tpu as pltpu
from jax.experimental.pallas import tpu_sc as plsc

VOCAB = 100000
BATCH = 1024
SEQLEN = 200
SEQP = 208
LANES = 16
NUM_CORES = 2
NUM_SUBCORES = 16
NW = NUM_CORES * NUM_SUBCORES
CPW = BATCH // NW
CHUNKS = SEQP // LANES
TAB = VOCAB + LANES


def _sc_body(text_hbm, w_hbm, b_hbm, out_hbm,
             tok_v, tab_v, w_v, b_v, out_v):
    wid = lax.axis_index("s") * NUM_CORES + lax.axis_index("c")
    base = wid * CPW

    pltpu.sync_copy(text_hbm.at[pl.ds(base * SEQP, CPW * SEQP)], tok_v)
    pltpu.sync_copy(w_hbm.at[pl.ds(0, LANES)], w_v)
    pltpu.sync_copy(b_hbm, b_v)

    iota = lax.iota(jnp.int32, LANES)

    def one_col(j):
        cbase = j * SEQP
        toks, tags = [], []
        for k in range(CHUNKS):
            tok = tok_v[pl.ds(cbase + k * LANES, LANES)]
            tag = cbase + k * LANES + iota
            toks.append(tok)
            tags.append(tag)
            plsc.store_scatter(tab_v, [tok], tag)
        cnt = jnp.zeros((LANES,), jnp.int32)
        for k in range(CHUNKS):
            q = plsc.load_gather(tab_v, [toks[k]])
            cnt = cnt + jnp.where(q == tags[k], 1, 0).astype(jnp.int32)
        return jnp.sum(cnt) - (SEQP - SEQLEN)

    def col_body(i, nv):
        j = 2 * i
        n_a = one_col(j)
        n_b = one_col(j + 1)
        upd = (jnp.where(iota == j % LANES, n_a, 0)
               + jnp.where(iota == (j + 1) % LANES, n_b, 0)).astype(jnp.int32)
        zero = jnp.zeros((LANES,), jnp.int32)
        in_lo = j < LANES
        nv0 = nv[0] + jnp.where(in_lo, upd, zero)
        nv1 = nv[1] + jnp.where(in_lo, zero, upd)
        return (nv0, nv1)

    zero16 = jnp.zeros((LANES,), jnp.int32)
    nvecs = lax.fori_loop(0, CPW // 2, col_body, (zero16, zero16))

    wvec = w_v[...]
    w0 = jnp.full((LANES,), wvec[0], jnp.float32)
    w1 = jnp.full((LANES,), wvec[1], jnp.float32)
    bv = b_v[...]
    for g in range(CPW // LANES):
        nvec = nvecs[g].astype(jnp.float32)
        y = nvec * w1 + (jnp.float32(VOCAB) - nvec) * w0 + bv
        ge = jnp.where(y >= 0.0, 1.0, 0.0).astype(jnp.float32)
        out_v[pl.ds(g * LANES, LANES)] = ge
        out_v[pl.ds(CPW + g * LANES, LANES)] = 1.0 - ge

    pltpu.sync_copy(out_v.at[pl.ds(0, CPW)], out_hbm.at[pl.ds(base, CPW)])
    pltpu.sync_copy(out_v.at[pl.ds(CPW, CPW)],
                    out_hbm.at[pl.ds(BATCH + base, CPW)])


@functools.cache
def _sc_kernel():
    return pl.kernel(
        _sc_body,
        mesh=plsc.VectorSubcoreMesh(core_axis_name="c", subcore_axis_name="s"),
        compiler_params=pltpu.CompilerParams(needs_layout_passes=False),
        out_type=jax.ShapeDtypeStruct((2 * BATCH,), jnp.float32),
        scratch_types=[
            pltpu.VMEM((CPW * SEQP,), jnp.int32),
            pltpu.VMEM((TAB,), jnp.int32),
            pltpu.VMEM((LANES,), jnp.float32),
            pltpu.VMEM((LANES,), jnp.float32),
            pltpu.VMEM((2 * CPW,), jnp.float32),
        ],
    )


def kernel(text, W, b):
    text_t = jnp.transpose(text).astype(jnp.int32)
    pad = jnp.broadcast_to(
        jnp.arange(VOCAB, VOCAB + (SEQP - SEQLEN), dtype=jnp.int32)[None, :],
        (BATCH, SEQP - SEQLEN))
    text_flat = jnp.concatenate([text_t, pad], axis=1).reshape(-1)
    b_vec = jnp.full((LANES,), b, jnp.float32)
    out_flat = _sc_kernel()(text_flat, W.astype(jnp.float32), b_vec)
    return out_flat.reshape(2, BATCH).astype(jnp.bool_)


# speedup vs baseline: 1.0129x; 1.0129x over previous
import functools

import jax
import jax.numpy as jnp
from jax import lax
from jax.experimental import pallas as pl
from jax.experimental.pallas import tpu as pltpu
from jax.experimental.pallas import tpu_sc as plsc

VOCAB = 100000
BATCH = 1024
SEQLEN = 200
SEQP = 208
LANES = 16
NUM_CORES = 2
NUM_SUBCORES = 16
NW = NUM_CORES * NUM_SUBCORES
CPW = BATCH // NW
CHUNKS = SEQP // LANES
TAB = VOCAB + LANES


def _sc_body(text_hbm, w_hbm, b_hbm, out_hbm,
             tok_v, tab_v, w_v, b_v, out_v):
    wid = lax.axis_index("s") * NUM_CORES + lax.axis_index("c")
    base = wid * CPW

    pltpu.sync_copy(text_hbm.at[pl.ds(base * SEQP, CPW * SEQP)], tok_v)
    pltpu.sync_copy(w_hbm.at[pl.ds(0, LANES)], w_v)
    pltpu.sync_copy(b_hbm, b_v)

    iota = lax.iota(jnp.int32, LANES)

    def col_body(j, nv):
        cbase = j * SEQP
        toks, tags = [], []
        for k in range(CHUNKS):
            tok = tok_v[pl.ds(cbase + k * LANES, LANES)]
            tag = cbase + k * LANES + iota
            toks.append(tok)
            tags.append(tag)
            plsc.store_scatter(tab_v, [tok], tag)
        cnt = jnp.zeros((LANES,), jnp.int32)
        for k in range(CHUNKS):
            q = plsc.load_gather(tab_v, [toks[k]])
            cnt = cnt + jnp.where(q == tags[k], 1, 0).astype(jnp.int32)
        n = jnp.sum(cnt) - (SEQP - SEQLEN)
        upd = jnp.where(iota == j % LANES, n, 0).astype(jnp.int32)
        zero = jnp.zeros((LANES,), jnp.int32)
        in_lo = j < LANES
        nv0 = nv[0] + jnp.where(in_lo, upd, zero)
        nv1 = nv[1] + jnp.where(in_lo, zero, upd)
        return (nv0, nv1)

    zero16 = jnp.zeros((LANES,), jnp.int32)
    nvecs = lax.fori_loop(0, CPW, col_body, (zero16, zero16))

    wvec = w_v[...]
    w0 = jnp.full((LANES,), wvec[0], jnp.float32)
    w1 = jnp.full((LANES,), wvec[1], jnp.float32)
    bv = b_v[...]
    for g in range(CPW // LANES):
        nvec = nvecs[g].astype(jnp.float32)
        y = nvec * w1 + (jnp.float32(VOCAB) - nvec) * w0 + bv
        ge = jnp.where(y >= 0.0, 1.0, 0.0).astype(jnp.float32)
        out_v[pl.ds(g * LANES, LANES)] = ge
        out_v[pl.ds(CPW + g * LANES, LANES)] = 1.0 - ge

    pltpu.sync_copy(out_v.at[pl.ds(0, CPW)], out_hbm.at[pl.ds(base, CPW)])
    pltpu.sync_copy(out_v.at[pl.ds(CPW, CPW)],
                    out_hbm.at[pl.ds(BATCH + base, CPW)])


@functools.cache
def _sc_kernel():
    return pl.kernel(
        _sc_body,
        mesh=plsc.VectorSubcoreMesh(core_axis_name="c", subcore_axis_name="s"),
        compiler_params=pltpu.CompilerParams(needs_layout_passes=False),
        out_type=jax.ShapeDtypeStruct((2 * BATCH,), jnp.float32),
        scratch_types=[
            pltpu.VMEM((CPW * SEQP,), jnp.int32),
            pltpu.VMEM((TAB,), jnp.int32),
            pltpu.VMEM((LANES,), jnp.float32),
            pltpu.VMEM((LANES,), jnp.float32),
            pltpu.VMEM((2 * CPW,), jnp.float32),
        ],
    )


def kernel(text, W, b):
    text_t = jnp.transpose(text).astype(jnp.int32)
    pad = jnp.broadcast_to(
        jnp.arange(VOCAB, VOCAB + (SEQP - SEQLEN), dtype=jnp.int32)[None, :],
        (BATCH, SEQP - SEQLEN))
    text_flat = jnp.concatenate([text_t, pad], axis=1).reshape(-1)
    b_vec = jnp.full((LANES,), b, jnp.float32)
    out_flat = _sc_kernel()(text_flat, W.astype(jnp.float32), b_vec)
    return out_flat.reshape(2, BATCH).astype(jnp.bool_)
